# baseline (speedup 1.0000x reference)
import jax
import jax.numpy as jnp
from jax.experimental import pallas as pl
from jax.experimental.pallas import tpu as pltpu

_B, _H, _W, _C = 2, 384, 384, 96
_RP = 8
_S = _C * _W // 128
_L = 128


def _unpool_mask_body(x_ref, o_ref):
    xe = x_ref[0, :, 0]
    xo = x_ref[0, :, 1]

    even_w = (jax.lax.broadcasted_iota(jnp.int32, xe.shape, 2) & 1) == 0

    def pair_swap(a):
        return jnp.where(
            even_w,
            pltpu.roll(a, _L - 1, axis=2),
            pltpu.roll(a, 1, axis=2),
        )

    pe = pair_swap(xe)
    po = pair_swap(xo)

    win_e = (xe > pe) | (even_w & (xe == pe))
    win_o = (xo > po) | (even_w & (xo == po))

    rv_e = jnp.maximum(xe, pe)
    rv_o = jnp.maximum(xo, po)
    e_ge = rv_e >= rv_o

    surv_e = win_e & e_ge
    surv_o = win_o & ~e_ge

    o_ref[0, :, 0] = jnp.where(surv_e, xe, 0.0)
    o_ref[0, :, 1] = jnp.where(surv_o, xo, 0.0)


def kernel(x):
    xt = jnp.transpose(x, (0, 1, 3, 2))
    x5 = xt.reshape(_B, _H // 2, 2, _S, _L)
    out = pl.pallas_call(
        _unpool_mask_body,
        grid=(_B, (_H // 2) // _RP),
        in_specs=[pl.BlockSpec((1, _RP, 2, _S, _L), lambda b, i: (b, i, 0, 0, 0))],
        out_specs=pl.BlockSpec((1, _RP, 2, _S, _L), lambda b, i: (b, i, 0, 0, 0)),
        out_shape=jax.ShapeDtypeStruct((_B, _H // 2, 2, _S, _L), x.dtype),
    )(x5)
    return jnp.transpose(out.reshape(_B, _H, _C, _W), (0, 1, 3, 2))


# speedup vs baseline: 2.7931x; 2.7931x over previous
import jax
import jax.numpy as jnp
from jax.experimental import pallas as pl
from jax.experimental.pallas import tpu as pltpu

_B, _H, _W, _C = 2, 384, 384, 96
_RP = 8


def _unpool_mask_body(x_ref, o_ref):
    xe = x_ref[0, :, 0]
    xo = x_ref[0, :, 1]
    w = xe.shape[2]

    even_w = (jax.lax.broadcasted_iota(jnp.int32, xe.shape, 2) & 1) == 0

    def pair_swap(a):
        return jnp.where(
            even_w,
            pltpu.roll(a, w - 1, axis=2),
            pltpu.roll(a, 1, axis=2),
        )

    pe = pair_swap(xe)
    po = pair_swap(xo)

    win_e = (xe > pe) | (even_w & (xe == pe))
    win_o = (xo > po) | (even_w & (xo == po))

    e_ge = jnp.maximum(xe, pe) >= jnp.maximum(xo, po)

    o_ref[0, :, 0] = jnp.where(win_e & e_ge, xe, 0.0)
    o_ref[0, :, 1] = jnp.where(win_o & ~e_ge, xo, 0.0)


def kernel(x):
    xt = jnp.transpose(x, (0, 1, 3, 2))
    x5 = xt.reshape(_B, _H // 2, 2, _C, _W)
    out = pl.pallas_call(
        _unpool_mask_body,
        grid=(_B, (_H // 2) // _RP),
        in_specs=[pl.BlockSpec((1, _RP, 2, _C, _W), lambda b, i: (b, i, 0, 0, 0))],
        out_specs=pl.BlockSpec((1, _RP, 2, _C, _W), lambda b, i: (b, i, 0, 0, 0)),
        out_shape=jax.ShapeDtypeStruct((_B, _H // 2, 2, _C, _W), x.dtype),
    )(x5)
    return jnp.transpose(out.reshape(_B, _H, _C, _W), (0, 1, 3, 2))


# speedup vs baseline: 2.9609x; 1.0601x over previous
import jax
import jax.numpy as jnp
from jax.experimental import pallas as pl
from jax.experimental.pallas import tpu as pltpu

_B, _H, _W, _C = 2, 384, 384, 96
_RP = 16


def _unpool_mask_body(x_ref, o_ref):
    xe = x_ref[0, :, 0]
    xo = x_ref[0, :, 1]
    w = xe.shape[2]

    even_w = (jax.lax.broadcasted_iota(jnp.int32, xe.shape, 2) & 1) == 0

    def pair_swap(a):
        return jnp.where(
            even_w,
            pltpu.roll(a, w - 1, axis=2),
            pltpu.roll(a, 1, axis=2),
        )

    pe = pair_swap(xe)
    po = pair_swap(xo)

    win_e = (xe > pe) | (even_w & (xe == pe))
    win_o = (xo > po) | (even_w & (xo == po))

    e_ge = jnp.maximum(xe, pe) >= jnp.maximum(xo, po)

    o_ref[0, :, 0] = jnp.where(win_e & e_ge, xe, 0.0)
    o_ref[0, :, 1] = jnp.where(win_o & ~e_ge, xo, 0.0)


def kernel(x):
    xt = jnp.transpose(x, (0, 1, 3, 2))
    x5 = xt.reshape(_B, _H // 2, 2, _C, _W)
    out = pl.pallas_call(
        _unpool_mask_body,
        grid=(_B, (_H // 2) // _RP),
        in_specs=[pl.BlockSpec((1, _RP, 2, _C, _W), lambda b, i: (b, i, 0, 0, 0))],
        out_specs=pl.BlockSpec((1, _RP, 2, _C, _W), lambda b, i: (b, i, 0, 0, 0)),
        out_shape=jax.ShapeDtypeStruct((_B, _H // 2, 2, _C, _W), x.dtype),
    )(x5)
    return jnp.transpose(out.reshape(_B, _H, _C, _W), (0, 1, 3, 2))
